# Initial kernel scaffold; baseline (speedup 1.0000x reference)
import jax
import jax.numpy as jnp
from jax.experimental import pallas as pl


def kernel(x, pos_table):
    raise NotImplementedError("write your pallas kernel here")


# speedup vs baseline: 3.5814x; 3.5814x over previous
import functools

import jax
import jax.numpy as jnp
from jax import lax
from jax.experimental import pallas as pl
from jax.experimental.pallas import tpu as pltpu
from jax.experimental.pallas import tpu_sc as plsc

_NUM_CORES = 2
_NUM_SUBCORES = 16
_NUM_WORKERS = _NUM_CORES * _NUM_SUBCORES


@functools.lru_cache(maxsize=None)
def _broadcast_kernel(batch, seq, hidden):
    rows_per_worker = seq // _NUM_WORKERS
    chunk = min(rows_per_worker, 64)
    num_chunks = rows_per_worker // chunk
    mesh = plsc.VectorSubcoreMesh(core_axis_name="c", subcore_axis_name="s")

    @functools.partial(
        pl.kernel,
        mesh=mesh,
        out_type=jax.ShapeDtypeStruct((batch, seq, hidden), jnp.float32),
        scratch_types=[
            pltpu.VMEM((chunk, hidden), jnp.float32),
            pltpu.SemaphoreType.DMA,
        ],
    )
    def k(table_hbm, out_hbm, buf, sem):
        wid = lax.axis_index("s") * _NUM_CORES + lax.axis_index("c")
        base = wid * rows_per_worker
        for i in range(num_chunks):
            row0 = base + i * chunk
            pltpu.sync_copy(table_hbm.at[pl.ds(row0, chunk), :], buf)
            handles = [
                pltpu.async_copy(buf, out_hbm.at[b, pl.ds(row0, chunk), :], sem)
                for b in range(batch)
            ]
            for h in handles:
                h.wait()

    return k


def kernel(x, pos_table):
    batch = x.shape[0]
    seq, hidden = pos_table.shape
    return _broadcast_kernel(batch, seq, hidden)(pos_table)
